# Initial kernel scaffold; baseline (speedup 1.0000x reference)
import functools
import jax
import jax.numpy as jnp
from jax import lax
from jax.experimental import pallas as pl
from jax.experimental.pallas import tpu as pltpu, tpu_sc as plsc

B = 32768
D0, D1, D2 = 16, 56, 56
DTOT = D0 + D1 + D2

NC, NS = 2, 16
NW = NC * NS
B_PER_W = B // NW
CHUNK = 128
N_CHUNK = B_PER_W // CHUNK


def _sc_body(idx_hbm, t0_hbm, t1_hbm, t2_hbm, out_hbm,
             idx_v, v0, v1, v2, sem_idx, sem_g):
    wid = lax.axis_index("s") * NC + lax.axis_index("c")
    pltpu.async_copy(idx_hbm.at[wid], idx_v, sem_idx).wait()

    def chunk_body(c, carry):
        base = wid * B_PER_W + c * CHUNK
        g0 = pltpu.async_copy(t0_hbm.at[idx_v.at[0, c]], v0, sem_g)
        g1 = pltpu.async_copy(t1_hbm.at[idx_v.at[1, c]], v1, sem_g)
        g2 = pltpu.async_copy(t2_hbm.at[idx_v.at[2, c]], v2, sem_g)
        g0.wait()
        g1.wait()
        g2.wait()
        pltpu.sync_copy(v0, out_hbm.at[pl.ds(base, CHUNK), pl.ds(0, D0)])
        pltpu.sync_copy(v1, out_hbm.at[pl.ds(base, CHUNK), pl.ds(D0, D1)])
        pltpu.sync_copy(v2, out_hbm.at[pl.ds(base, CHUNK), pl.ds(D0 + D1, D2)])
        return carry

    lax.fori_loop(0, N_CHUNK, chunk_body, 0)


@jax.jit
def _rope_gather(idx, t0, t1, t2):
    mesh = plsc.VectorSubcoreMesh(core_axis_name="c", subcore_axis_name="s")
    f = pl.kernel(
        _sc_body,
        out_type=jax.ShapeDtypeStruct((B, DTOT), jnp.float32),
        mesh=mesh,
        scratch_types=[
            pltpu.VMEM((3, N_CHUNK, CHUNK), jnp.int32),
            pltpu.VMEM((CHUNK, D0), jnp.float32),
            pltpu.VMEM((CHUNK, D1), jnp.float32),
            pltpu.VMEM((CHUNK, D2), jnp.float32),
            pltpu.SemaphoreType.DMA,
            pltpu.SemaphoreType.DMA,
        ],
    )
    return f(idx, t0, t1, t2)


def kernel(ids, freqs_0, freqs_1, freqs_2):
    idx = ids.astype(jnp.int32).T.reshape(3, NW, N_CHUNK, CHUNK)
    idx = idx.transpose(1, 0, 2, 3)
    t0 = freqs_0.reshape(-1, D0)
    t1 = freqs_1.reshape(-1, D1)
    t2 = freqs_2.reshape(-1, D2)
    out = _rope_gather(idx, t0, t1, t2)
    return out.reshape(B, DTOT // 2, 2)


# speedup vs baseline: 11.5321x; 11.5321x over previous
import functools
import jax
import jax.numpy as jnp
from jax import lax
from jax.experimental import pallas as pl
from jax.experimental.pallas import tpu as pltpu, tpu_sc as plsc

B = 32768
D0, D1, D2 = 16, 56, 56
DTOT = D0 + D1 + D2

NC, NS = 2, 16
NW = NC * NS
B_PER_W = B // NW
CHUNK = 128
N_CHUNK = B_PER_W // CHUNK


def _sc_body(idx_hbm, t0_hbm, t1_hbm, t2_hbm, out_hbm,
             idx_v, comb, sem_idx, sem_g):
    wid = lax.axis_index("s") * NC + lax.axis_index("c")
    pltpu.async_copy(idx_hbm.at[wid], idx_v, sem_idx).wait()

    def chunk_body(c, carry):
        base = wid * B_PER_W + c * CHUNK
        pltpu.async_copy(t0_hbm.at[idx_v.at[0, c]], comb, sem_g).wait()
        pltpu.async_copy(t1_hbm.at[idx_v.at[1, c]], comb, sem_g,
                         add=True).wait()
        pltpu.async_copy(t2_hbm.at[idx_v.at[2, c]], comb, sem_g,
                         add=True).wait()
        pltpu.sync_copy(comb, out_hbm.at[pl.ds(base, CHUNK), :])
        return carry

    lax.fori_loop(0, N_CHUNK, chunk_body, 0)


@jax.jit
def _rope_gather(idx, t0, t1, t2):
    mesh = plsc.VectorSubcoreMesh(core_axis_name="c", subcore_axis_name="s")
    f = pl.kernel(
        _sc_body,
        out_type=jax.ShapeDtypeStruct((B, DTOT), jnp.float32),
        mesh=mesh,
        scratch_types=[
            pltpu.VMEM((3, N_CHUNK, CHUNK), jnp.int32),
            pltpu.VMEM((CHUNK, DTOT), jnp.float32),
            pltpu.SemaphoreType.DMA,
            pltpu.SemaphoreType.DMA,
        ],
    )
    return f(idx, t0, t1, t2)


def kernel(ids, freqs_0, freqs_1, freqs_2):
    idx = ids.astype(jnp.int32).T.reshape(3, NW, N_CHUNK, CHUNK)
    idx = idx.transpose(1, 0, 2, 3)
    V = 512
    t0 = freqs_0[:V].reshape(V, D0)
    t1 = freqs_1[:V].reshape(V, D1)
    t2 = freqs_2[:V].reshape(V, D2)
    z0 = jnp.zeros((V, D0), jnp.float32)
    z1 = jnp.zeros((V, D1), jnp.float32)
    z2 = jnp.zeros((V, D2), jnp.float32)
    t0p = jnp.concatenate([t0, z1, z2], axis=1)
    t1p = jnp.concatenate([z0, t1, z2], axis=1)
    t2p = jnp.concatenate([z0, z1, t2], axis=1)
    out = _rope_gather(idx, t0p, t1p, t2p)
    return out.reshape(B, DTOT // 2, 2)


# speedup vs baseline: 12.2443x; 1.0618x over previous
import functools
import jax
import jax.numpy as jnp
from jax import lax
from jax.experimental import pallas as pl
from jax.experimental.pallas import tpu as pltpu, tpu_sc as plsc

B = 32768
D0, D1, D2 = 16, 56, 56
DTOT = D0 + D1 + D2
V = 512

NC, NS = 2, 16
NW = NC * NS
B_PER_W = B // NW
CHUNK = 128
N_CHUNK = B_PER_W // CHUNK
DEPTH = 4


def _sc_body(idx_hbm, t0_hbm, t1_hbm, t2_hbm, out_hbm,
             idx_v, combs, sem_in, sems_g, sems_a, sems_w):
    wid = lax.axis_index("s") * NC + lax.axis_index("c")
    pltpu.async_copy(idx_hbm.at[wid], idx_v, sem_in).wait()

    g_pend = [None] * DEPTH
    a_pend = [None] * DEPTH
    w_pend = [None] * DEPTH

    def fire_g0(c):
        p = c % DEPTH
        if w_pend[p] is not None:
            w_pend[p].wait()
        g_pend[p] = pltpu.async_copy(
            t0_hbm.at[idx_v.at[0, c]], combs[p], sems_g[p])

    def fire_adds(c):
        p = c % DEPTH
        g_pend[p].wait()
        a_pend[p] = (
            pltpu.async_copy(t1_hbm.at[idx_v.at[1, c]], combs[p],
                             sems_a[p], add=True),
            pltpu.async_copy(t2_hbm.at[idx_v.at[2, c]], combs[p],
                             sems_a[p], add=True),
        )

    def fire_write(c):
        p = c % DEPTH
        a_pend[p][0].wait()
        a_pend[p][1].wait()
        base = wid * B_PER_W + c * CHUNK
        w_pend[p] = pltpu.async_copy(
            combs[p], out_hbm.at[pl.ds(base, CHUNK), :], sems_w[p])

    fire_g0(0)
    fire_g0(1)
    fire_adds(0)
    for c in range(2, N_CHUNK):
        fire_g0(c)
        fire_adds(c - 1)
        fire_write(c - 2)
    fire_adds(N_CHUNK - 1)
    fire_write(N_CHUNK - 2)
    fire_write(N_CHUNK - 1)
    for p in range(DEPTH):
        if w_pend[p] is not None:
            w_pend[p].wait()


@jax.jit
def _rope_gather(idx, t0, t1, t2):
    mesh = plsc.VectorSubcoreMesh(core_axis_name="c", subcore_axis_name="s")

    def body(idx_hbm, t0_hbm, t1_hbm, t2_hbm, out_hbm, idx_v,
             c0, c1, c2, c3, sem_in,
             g0, g1, g2, g3, a0, a1, a2, a3, w0, w1, w2, w3):
        _sc_body(idx_hbm, t0_hbm, t1_hbm, t2_hbm, out_hbm, idx_v,
                 (c0, c1, c2, c3), sem_in,
                 (g0, g1, g2, g3), (a0, a1, a2, a3), (w0, w1, w2, w3))

    f = pl.kernel(
        body,
        out_type=jax.ShapeDtypeStruct((B, DTOT), jnp.float32),
        mesh=mesh,
        scratch_types=[
            pltpu.VMEM((3, N_CHUNK, CHUNK), jnp.int32),
        ] + [pltpu.VMEM((CHUNK, DTOT), jnp.float32)] * DEPTH
          + [pltpu.SemaphoreType.DMA] * (1 + 3 * DEPTH),
    )
    return f(idx, t0, t1, t2)


def kernel(ids, freqs_0, freqs_1, freqs_2):
    idx = ids.astype(jnp.int32).T.reshape(3, NW, N_CHUNK, CHUNK)
    idx = idx.transpose(1, 0, 2, 3)
    t0 = freqs_0[:V].reshape(V, D0)
    t1 = freqs_1[:V].reshape(V, D1)
    t2 = freqs_2[:V].reshape(V, D2)
    z0 = jnp.zeros((V, D0), jnp.float32)
    z1 = jnp.zeros((V, D1), jnp.float32)
    z2 = jnp.zeros((V, D2), jnp.float32)
    t0p = jnp.concatenate([t0, z1, z2], axis=1)
    t1p = jnp.concatenate([z0, t1, z2], axis=1)
    t2p = jnp.concatenate([z0, z1, t2], axis=1)
    out = _rope_gather(idx, t0p, t1p, t2p)
    return out.reshape(B, DTOT // 2, 2)


# speedup vs baseline: 12.3640x; 1.0098x over previous
import functools
import jax
import jax.numpy as jnp
from jax import lax
from jax.experimental import pallas as pl
from jax.experimental.pallas import tpu as pltpu, tpu_sc as plsc

B = 32768
D0, D1, D2 = 16, 56, 56
DTOT = D0 + D1 + D2
V = 512

NC, NS = 1, 16
NW = NC * NS
B_PER_W = B // NW
CHUNK = 128
N_CHUNK = B_PER_W // CHUNK
DEPTH = 4


def _sc_body(idx_hbm, t0_hbm, t1_hbm, t2_hbm, out_hbm,
             idx_v, combs, sem_in, sems_g, sems_a, sems_w):
    wid = lax.axis_index("s") * NC + lax.axis_index("c")
    pltpu.async_copy(idx_hbm.at[wid], idx_v, sem_in).wait()

    g_pend = [None] * DEPTH
    a_pend = [None] * DEPTH
    w_pend = [None] * DEPTH

    def fire_g0(c):
        p = c % DEPTH
        if w_pend[p] is not None:
            w_pend[p].wait()
        g_pend[p] = pltpu.async_copy(
            t0_hbm.at[idx_v.at[0, c]], combs[p], sems_g[p])

    def fire_adds(c):
        p = c % DEPTH
        g_pend[p].wait()
        a_pend[p] = (
            pltpu.async_copy(t1_hbm.at[idx_v.at[1, c]], combs[p],
                             sems_a[p], add=True),
            pltpu.async_copy(t2_hbm.at[idx_v.at[2, c]], combs[p],
                             sems_a[p], add=True),
        )

    def fire_write(c):
        p = c % DEPTH
        a_pend[p][0].wait()
        a_pend[p][1].wait()
        base = wid * B_PER_W + c * CHUNK
        w_pend[p] = pltpu.async_copy(
            combs[p], out_hbm.at[pl.ds(base, CHUNK), :], sems_w[p])

    fire_g0(0)
    fire_g0(1)
    fire_adds(0)
    for c in range(2, N_CHUNK):
        fire_g0(c)
        fire_adds(c - 1)
        fire_write(c - 2)
    fire_adds(N_CHUNK - 1)
    fire_write(N_CHUNK - 2)
    fire_write(N_CHUNK - 1)
    for p in range(DEPTH):
        if w_pend[p] is not None:
            w_pend[p].wait()


@jax.jit
def _rope_gather(idx, t0, t1, t2):
    mesh = plsc.VectorSubcoreMesh(core_axis_name="c", subcore_axis_name="s",
                                  num_cores=NC)

    def body(idx_hbm, t0_hbm, t1_hbm, t2_hbm, out_hbm, idx_v,
             c0, c1, c2, c3, sem_in,
             g0, g1, g2, g3, a0, a1, a2, a3, w0, w1, w2, w3):
        _sc_body(idx_hbm, t0_hbm, t1_hbm, t2_hbm, out_hbm, idx_v,
                 (c0, c1, c2, c3), sem_in,
                 (g0, g1, g2, g3), (a0, a1, a2, a3), (w0, w1, w2, w3))

    f = pl.kernel(
        body,
        out_type=jax.ShapeDtypeStruct((B, DTOT), jnp.float32),
        mesh=mesh,
        scratch_types=[
            pltpu.VMEM((3, N_CHUNK, CHUNK), jnp.int32),
        ] + [pltpu.VMEM((CHUNK, DTOT), jnp.float32)] * DEPTH
          + [pltpu.SemaphoreType.DMA] * (1 + 3 * DEPTH),
    )
    return f(idx, t0, t1, t2)


def kernel(ids, freqs_0, freqs_1, freqs_2):
    idx = ids.astype(jnp.int32).T.reshape(3, NW, N_CHUNK, CHUNK)
    idx = idx.transpose(1, 0, 2, 3)
    t0 = freqs_0[:V].reshape(V, D0)
    t1 = freqs_1[:V].reshape(V, D1)
    t2 = freqs_2[:V].reshape(V, D2)
    z0 = jnp.zeros((V, D0), jnp.float32)
    z1 = jnp.zeros((V, D1), jnp.float32)
    z2 = jnp.zeros((V, D2), jnp.float32)
    t0p = jnp.concatenate([t0, z1, z2], axis=1)
    t1p = jnp.concatenate([z0, t1, z2], axis=1)
    t2p = jnp.concatenate([z0, z1, t2], axis=1)
    out = _rope_gather(idx, t0p, t1p, t2p)
    return out.reshape(B, DTOT // 2, 2)


# speedup vs baseline: 13.9382x; 1.1273x over previous
import functools
import jax
import jax.numpy as jnp
from jax import lax
from jax.experimental import pallas as pl
from jax.experimental.pallas import tpu as pltpu, tpu_sc as plsc

B = 32768
D0, D1, D2 = 16, 56, 56
DTOT = D0 + D1 + D2
V = 512

NC, NS = 2, 16
REPL = 8
NW = NC * NS
B_PER_W = B // NW
CHUNK = 128
N_CHUNK = B_PER_W // CHUNK
DEPTH = 4


def _sc_body(idx_hbm, t0_hbm, t1_hbm, t2_hbm, out_hbm,
             idx_v, combs, sem_in, sems_g, sems_a, sems_w):
    wid = lax.axis_index("s") * NC + lax.axis_index("c")
    pltpu.async_copy(idx_hbm.at[wid], idx_v, sem_in).wait()

    g_pend = [None] * DEPTH
    a_pend = [None] * DEPTH
    w_pend = [None] * DEPTH

    def fire_g0(c):
        p = c % DEPTH
        if w_pend[p] is not None:
            w_pend[p].wait()
        g_pend[p] = pltpu.async_copy(
            t0_hbm.at[idx_v.at[0, c]], combs[p], sems_g[p])

    def fire_adds(c):
        p = c % DEPTH
        g_pend[p].wait()
        a_pend[p] = (
            pltpu.async_copy(t1_hbm.at[idx_v.at[1, c]], combs[p],
                             sems_a[p], add=True),
            pltpu.async_copy(t2_hbm.at[idx_v.at[2, c]], combs[p],
                             sems_a[p], add=True),
        )

    def fire_write(c):
        p = c % DEPTH
        a_pend[p][0].wait()
        a_pend[p][1].wait()
        base = wid * B_PER_W + c * CHUNK
        w_pend[p] = pltpu.async_copy(
            combs[p], out_hbm.at[pl.ds(base, CHUNK), :], sems_w[p])

    fire_g0(0)
    fire_g0(1)
    fire_adds(0)
    for c in range(2, N_CHUNK):
        fire_g0(c)
        fire_adds(c - 1)
        fire_write(c - 2)
    fire_adds(N_CHUNK - 1)
    fire_write(N_CHUNK - 2)
    fire_write(N_CHUNK - 1)
    for p in range(DEPTH):
        if w_pend[p] is not None:
            w_pend[p].wait()


@jax.jit
def _rope_gather(idx, t0, t1, t2):
    mesh = plsc.VectorSubcoreMesh(core_axis_name="c", subcore_axis_name="s",
                                  num_cores=NC)

    def body(idx_hbm, t0_hbm, t1_hbm, t2_hbm, out_hbm, idx_v,
             c0, c1, c2, c3, sem_in,
             g0, g1, g2, g3, a0, a1, a2, a3, w0, w1, w2, w3):
        _sc_body(idx_hbm, t0_hbm, t1_hbm, t2_hbm, out_hbm, idx_v,
                 (c0, c1, c2, c3), sem_in,
                 (g0, g1, g2, g3), (a0, a1, a2, a3), (w0, w1, w2, w3))

    f = pl.kernel(
        body,
        out_type=jax.ShapeDtypeStruct((B, DTOT), jnp.float32),
        mesh=mesh,
        scratch_types=[
            pltpu.VMEM((3, N_CHUNK, CHUNK), jnp.int32),
        ] + [pltpu.VMEM((CHUNK, DTOT), jnp.float32)] * DEPTH
          + [pltpu.SemaphoreType.DMA] * (1 + 3 * DEPTH),
    )
    return f(idx, t0, t1, t2)


def kernel(ids, freqs_0, freqs_1, freqs_2):
    idx = ids.astype(jnp.int32).T.reshape(3, NW, N_CHUNK, CHUNK)
    idx = idx.transpose(1, 0, 2, 3)
    repl_off = (jnp.arange(NW, dtype=jnp.int32) % REPL) * V
    idx = idx + repl_off[:, None, None, None]
    t0 = freqs_0[:V].reshape(V, D0)
    t1 = freqs_1[:V].reshape(V, D1)
    t2 = freqs_2[:V].reshape(V, D2)
    z0 = jnp.zeros((V, D0), jnp.float32)
    z1 = jnp.zeros((V, D1), jnp.float32)
    z2 = jnp.zeros((V, D2), jnp.float32)
    t0p = jnp.tile(jnp.concatenate([t0, z1, z2], axis=1), (REPL, 1))
    t1p = jnp.tile(jnp.concatenate([z0, t1, z2], axis=1), (REPL, 1))
    t2p = jnp.tile(jnp.concatenate([z0, z1, t2], axis=1), (REPL, 1))
    out = _rope_gather(idx, t0p, t1p, t2p)
    return out.reshape(B, DTOT // 2, 2)
